# Initial kernel scaffold; baseline (speedup 1.0000x reference)
import jax
import jax.numpy as jnp
from jax.experimental import pallas as pl


def kernel(zz, zmap):
    raise NotImplementedError("write your pallas kernel here")


# speedup vs baseline: 15.7146x; 15.7146x over previous
import functools

import jax
import jax.numpy as jnp
from jax import lax
from jax.experimental import pallas as pl
from jax.experimental.pallas import tpu as pltpu
from jax.experimental.pallas import tpu_sc as plsc

_NROWS = 1_000_000
_NZ = 100
_R = 400
_CW = _R * _NZ
_NCH = _NROWS // _R
_G = _R // 16

_info = plsc.get_sparse_core_info()
_NC = _info.num_cores
_NW = _NC * _info.num_subcores
_NT_HI = -(-_NCH // _NW)
_NT_LO = _NCH // _NW
_N_HI = _NCH - _NT_LO * _NW
_PAIRS = -(-_NT_HI // 2)
_ZZW = _NT_HI * _R


@functools.partial(
    pl.kernel,
    out_type=jax.ShapeDtypeStruct((_NROWS * _NZ,), jnp.float32),
    mesh=plsc.VectorSubcoreMesh(core_axis_name="c", subcore_axis_name="s"),
    compiler_params=pltpu.CompilerParams(needs_layout_passes=False),
    scratch_types=[
        pltpu.VMEM((128,), jnp.int32),
        pltpu.VMEM((_ZZW,), jnp.int32),
        pltpu.VMEM((_CW,), jnp.float32),
        pltpu.VMEM((_CW,), jnp.float32),
        pltpu.VMEM((_R,), jnp.int32),
        pltpu.VMEM((_R,), jnp.int32),
        pltpu.SemaphoreType.DMA,
        pltpu.SemaphoreType.DMA,
    ],
)
def _sc_onehot(zz_hbm, zmap_hbm, out_hbm,
               zmap_v, zz_v, buf0, buf1, fl0, fl1, sem0, sem1):
    w = lax.axis_index("s") * _NC + lax.axis_index("c")
    bufs = (buf0, buf1)
    flats = (fl0, fl1)
    sems = (sem0, sem1)

    zeros16 = jnp.zeros((16,), jnp.float32)
    ones16 = jnp.ones((16,), jnp.float32)
    iota16 = lax.iota(jnp.int32, 16)

    pltpu.sync_copy(zmap_hbm, zmap_v)

    is_hi = w < _N_HI
    start = jnp.where(is_hi, _NT_HI * w, _NT_LO * w + _N_HI)
    nt = jnp.where(is_hi, _NT_HI, _NT_LO)

    pltpu.sync_copy(zz_hbm.at[pl.ds(start * _R, _NT_LO * _R)],
                    zz_v.at[pl.ds(0, _NT_LO * _R)])

    @pl.when(is_hi)
    def _():
        pltpu.sync_copy(zz_hbm.at[pl.ds(start * _R + _NT_LO * _R, _R)],
                        zz_v.at[pl.ds(_NT_LO * _R, _R)])

    def _zero(i, _):
        buf0[pl.ds(i * 16, 16)] = zeros16
        buf1[pl.ds(i * 16, 16)] = zeros16
        return 0
    lax.fori_loop(0, _CW // 16, _zero, 0)

    def _chunk(t, slot):
        @pl.when(t < nt)
        def _():
            buf, fl, sem = bufs[slot], flats[slot], sems[slot]
            out_dst = out_hbm.at[pl.ds((start + t) * _CW, _CW)]

            @pl.when(t >= 2)
            def _():
                pltpu.make_async_copy(buf, out_dst, sem).wait()
                for g in range(_G):
                    old = fl[pl.ds(g * 16, 16)]
                    plsc.store_scatter(buf, [old], zeros16)

            for g in range(_G):
                z = zz_v[pl.ds(t * _R + g * 16, 16)]
                idx = plsc.load_gather(zmap_v, [z])
                f = (iota16 + g * 16) * _NZ + idx
                fl[pl.ds(g * 16, 16)] = f
                plsc.store_scatter(buf, [f], ones16)

            pltpu.make_async_copy(buf, out_dst, sem).start()

    def _pair(p, _):
        _chunk(2 * p, 0)
        _chunk(2 * p + 1, 1)
        return 0
    lax.fori_loop(0, _PAIRS, _pair, 0)

    pltpu.make_async_copy(buf0, out_hbm.at[pl.ds(0, _CW)], sem0).wait()
    pltpu.make_async_copy(buf1, out_hbm.at[pl.ds(0, _CW)], sem1).wait()


def kernel(zz, zmap):
    zz = zz.reshape(-1).astype(jnp.int32)
    zmap_p = jnp.zeros((128,), jnp.int32).at[:_NZ].set(zmap.astype(jnp.int32))
    out = _sc_onehot(zz, zmap_p)
    return out.reshape(_NROWS, _NZ)


# speedup vs baseline: 22.2109x; 1.4134x over previous
import functools

import jax
import jax.numpy as jnp
from jax import lax
from jax.experimental import pallas as pl
from jax.experimental.pallas import tpu as pltpu
from jax.experimental.pallas import tpu_sc as plsc

_NROWS = 1_000_000
_NZ = 100
_R = 400
_NCH = _NROWS // _R
_G = _R // 16
_NPAD = 1 << 20

_info = plsc.get_sparse_core_info()
_NC = _info.num_cores
_NW = _NC * _info.num_subcores
_NT_HI = -(-_NCH // _NW)
_NT_LO = _NCH // _NW
_N_HI = _NCH - _NT_LO * _NW
_ZZW = _NT_HI * _R


@functools.partial(
    pl.kernel,
    out_type=jax.ShapeDtypeStruct((_NPAD,), jnp.int32),
    mesh=plsc.VectorSubcoreMesh(core_axis_name="c", subcore_axis_name="s"),
    compiler_params=pltpu.CompilerParams(needs_layout_passes=False),
    scratch_types=[
        pltpu.VMEM((128,), jnp.int32),
        pltpu.VMEM((_ZZW,), jnp.int32),
        pltpu.VMEM((_ZZW,), jnp.int32),
    ],
)
def _sc_gather(zz_hbm, zmap_hbm, idx_hbm, zmap_v, zz_v, ibuf):
    w = lax.axis_index("s") * _NC + lax.axis_index("c")

    pltpu.sync_copy(zmap_hbm, zmap_v)

    is_hi = w < _N_HI
    start = jnp.where(is_hi, _NT_HI * w, _NT_LO * w + _N_HI)
    lo_words = _NT_LO * _R

    pltpu.sync_copy(zz_hbm.at[pl.ds(start * _R, lo_words)],
                    zz_v.at[pl.ds(0, lo_words)])

    @pl.when(is_hi)
    def _():
        pltpu.sync_copy(zz_hbm.at[pl.ds(start * _R + lo_words, _R)],
                        zz_v.at[pl.ds(lo_words, _R)])

    def _group(j, _):
        z = zz_v[pl.ds(j * 16, 16)]
        ibuf[pl.ds(j * 16, 16)] = plsc.load_gather(zmap_v, [z])
        return 0
    lax.fori_loop(0, lo_words // 16, _group, 0)

    @pl.when(is_hi)
    def _():
        for g in range(_G):
            z = zz_v[pl.ds(lo_words + g * 16, 16)]
            ibuf[pl.ds(lo_words + g * 16, 16)] = plsc.load_gather(zmap_v, [z])

    pltpu.sync_copy(ibuf.at[pl.ds(0, lo_words)],
                    idx_hbm.at[pl.ds(start * _R, lo_words)])

    @pl.when(is_hi)
    def _():
        pltpu.sync_copy(ibuf.at[pl.ds(lo_words, _R)],
                        idx_hbm.at[pl.ds(start * _R + lo_words, _R)])


_BLKR = 1024
_TGRID = -(-_NROWS // _BLKR)


def _tc_body(idx_ref, out_ref):
    t = jnp.transpose(idx_ref[...])
    lane_iota = lax.broadcasted_iota(jnp.int32, (128, _NZ), 1)
    for s in range(8):
        oh = (t[:, s:s + 1] == lane_iota).astype(jnp.float32)
        out_ref[pl.ds(s * 128, 128), :] = oh


def kernel(zz, zmap):
    zz = zz.reshape(-1).astype(jnp.int32)
    zmap_p = jnp.zeros((128,), jnp.int32).at[:_NZ].set(zmap.astype(jnp.int32))
    idx = _sc_gather(zz, zmap_p)
    out = pl.pallas_call(
        _tc_body,
        grid=(_TGRID,),
        in_specs=[pl.BlockSpec((8, 128), lambda i: (i, 0))],
        out_specs=pl.BlockSpec((_BLKR, _NZ), lambda i: (i, 0)),
        out_shape=jax.ShapeDtypeStruct((_NROWS, _NZ), jnp.float32),
    )(idx.reshape(_NPAD // 128, 128))
    return out


# speedup vs baseline: 22.5618x; 1.0158x over previous
import functools

import jax
import jax.numpy as jnp
from jax import lax
from jax.experimental import pallas as pl
from jax.experimental.pallas import tpu as pltpu
from jax.experimental.pallas import tpu_sc as plsc

_NROWS = 1_000_000
_NZ = 100
_R = 400
_NCH = _NROWS // _R
_G = _R // 16
_NPAD = 1 << 20

_info = plsc.get_sparse_core_info()
_NC = _info.num_cores
_NW = _NC * _info.num_subcores
_NT_HI = -(-_NCH // _NW)
_NT_LO = _NCH // _NW
_N_HI = _NCH - _NT_LO * _NW
_ZZW = _NT_HI * _R


@functools.partial(
    pl.kernel,
    out_type=jax.ShapeDtypeStruct((_NPAD,), jnp.int32),
    mesh=plsc.VectorSubcoreMesh(core_axis_name="c", subcore_axis_name="s"),
    compiler_params=pltpu.CompilerParams(needs_layout_passes=False),
    scratch_types=[
        pltpu.VMEM((128,), jnp.int32),
        pltpu.VMEM((_ZZW,), jnp.int32),
        pltpu.VMEM((_ZZW,), jnp.int32),
    ],
)
def _sc_gather(zz_hbm, zmap_hbm, idx_hbm, zmap_v, zz_v, ibuf):
    w = lax.axis_index("s") * _NC + lax.axis_index("c")

    pltpu.sync_copy(zmap_hbm, zmap_v)

    is_hi = w < _N_HI
    start = jnp.where(is_hi, _NT_HI * w, _NT_LO * w + _N_HI)
    lo_words = _NT_LO * _R

    pltpu.sync_copy(zz_hbm.at[pl.ds(start * _R, lo_words)],
                    zz_v.at[pl.ds(0, lo_words)])

    @pl.when(is_hi)
    def _():
        pltpu.sync_copy(zz_hbm.at[pl.ds(start * _R + lo_words, _R)],
                        zz_v.at[pl.ds(lo_words, _R)])

    def _group(j, _):
        z = zz_v[pl.ds(j * 16, 16)]
        ibuf[pl.ds(j * 16, 16)] = plsc.load_gather(zmap_v, [z])
        return 0
    lax.fori_loop(0, lo_words // 16, _group, 0)

    @pl.when(is_hi)
    def _():
        for g in range(_G):
            z = zz_v[pl.ds(lo_words + g * 16, 16)]
            ibuf[pl.ds(lo_words + g * 16, 16)] = plsc.load_gather(zmap_v, [z])

    pltpu.sync_copy(ibuf.at[pl.ds(0, lo_words)],
                    idx_hbm.at[pl.ds(start * _R, lo_words)])

    @pl.when(is_hi)
    def _():
        pltpu.sync_copy(ibuf.at[pl.ds(lo_words, _R)],
                        idx_hbm.at[pl.ds(start * _R + lo_words, _R)])


_BLKR = 1024
_TGRID = -(-_NROWS // _BLKR)


def _tc_body(idx_ref, out_ref):
    nsub = _BLKR // 128
    uf = idx_ref[...].astype(jnp.float32)
    lane_iota = lax.broadcasted_iota(jnp.int32, (128, _NZ), 1).astype(jnp.float32)
    sub_iota = lax.broadcasted_iota(jnp.int32, (nsub, _NZ), 0)
    for s in range(nsub):
        sel = (sub_iota == s).astype(jnp.float32)
        bc = lax.dot_general(uf, sel, (((0,), (0,)), ((), ())),
                             preferred_element_type=jnp.float32)
        oh = (bc == lane_iota).astype(jnp.float32)
        out_ref[pl.ds(s * 128, 128), :] = oh


def kernel(zz, zmap):
    zz = zz.reshape(-1).astype(jnp.int32)
    zmap_p = jnp.zeros((128,), jnp.int32).at[:_NZ].set(zmap.astype(jnp.int32))
    idx = _sc_gather(zz, zmap_p)
    out = pl.pallas_call(
        _tc_body,
        grid=(_TGRID,),
        in_specs=[pl.BlockSpec((_BLKR // 128, 128), lambda i: (i, 0))],
        out_specs=pl.BlockSpec((_BLKR, _NZ), lambda i: (i, 0)),
        out_shape=jax.ShapeDtypeStruct((_NROWS, _NZ), jnp.float32),
    )(idx.reshape(_NPAD // 128, 128))
    return out


# speedup vs baseline: 24.9184x; 1.1045x over previous
import functools

import jax
import jax.numpy as jnp
from jax import lax
from jax.experimental import pallas as pl
from jax.experimental.pallas import tpu as pltpu
from jax.experimental.pallas import tpu_sc as plsc

_NROWS = 1_000_000
_NZ = 100
_R = 400
_NCH = _NROWS // _R
_G = _R // 16
_NPAD = 1 << 20

_info = plsc.get_sparse_core_info()
_NC = _info.num_cores
_NW = _NC * _info.num_subcores
_NT_HI = -(-_NCH // _NW)
_NT_LO = _NCH // _NW
_N_HI = _NCH - _NT_LO * _NW
_ZZW = _NT_HI * _R


@functools.partial(
    pl.kernel,
    out_type=jax.ShapeDtypeStruct((_NPAD,), jnp.int32),
    mesh=plsc.VectorSubcoreMesh(core_axis_name="c", subcore_axis_name="s"),
    compiler_params=pltpu.CompilerParams(needs_layout_passes=False),
    scratch_types=[
        pltpu.VMEM((128,), jnp.int32),
        pltpu.VMEM((_ZZW,), jnp.int32),
        pltpu.VMEM((_ZZW,), jnp.int32),
    ],
)
def _sc_gather(zz_hbm, zmap_hbm, idx_hbm, zmap_v, zz_v, ibuf):
    w = lax.axis_index("s") * _NC + lax.axis_index("c")

    pltpu.sync_copy(zmap_hbm, zmap_v)

    is_hi = w < _N_HI
    start = jnp.where(is_hi, _NT_HI * w, _NT_LO * w + _N_HI)
    lo_words = _NT_LO * _R

    pltpu.sync_copy(zz_hbm.at[pl.ds(start * _R, lo_words)],
                    zz_v.at[pl.ds(0, lo_words)])

    @pl.when(is_hi)
    def _():
        pltpu.sync_copy(zz_hbm.at[pl.ds(start * _R + lo_words, _R)],
                        zz_v.at[pl.ds(lo_words, _R)])

    def _group(j, _):
        z = zz_v[pl.ds(j * 16, 16)]
        ibuf[pl.ds(j * 16, 16)] = plsc.load_gather(zmap_v, [z])
        return 0
    lax.fori_loop(0, lo_words // 16, _group, 0)

    @pl.when(is_hi)
    def _():
        for g in range(_G):
            z = zz_v[pl.ds(lo_words + g * 16, 16)]
            ibuf[pl.ds(lo_words + g * 16, 16)] = plsc.load_gather(zmap_v, [z])

    pltpu.sync_copy(ibuf.at[pl.ds(0, lo_words)],
                    idx_hbm.at[pl.ds(start * _R, lo_words)])

    @pl.when(is_hi)
    def _():
        pltpu.sync_copy(ibuf.at[pl.ds(lo_words, _R)],
                        idx_hbm.at[pl.ds(start * _R + lo_words, _R)])


_BLKR = 1024
_TGRID = -(-_NROWS // _BLKR)
_TAIL = _NROWS - (_TGRID - 1) * _BLKR
_NBUF = 8


def _tc_body(idx_ref, out_hbm, buf, sems):
    nsub = _BLKR // 128
    i = pl.program_id(0)
    j = lax.rem(i, _NBUF)
    uf = idx_ref[...].astype(jnp.float32)
    lane_iota = lax.broadcasted_iota(jnp.int32, (128, _NZ), 1).astype(jnp.float32)
    sub_iota = lax.broadcasted_iota(jnp.int32, (nsub, _NZ), 0)

    @pl.when(i >= _NBUF)
    def _():
        pltpu.make_async_copy(buf.at[j], out_hbm.at[pl.ds(0, _BLKR), :],
                              sems.at[j]).wait()

    for s in range(nsub):
        sel = (sub_iota == s).astype(jnp.float32)
        bc = lax.dot_general(uf, sel, (((0,), (0,)), ((), ())),
                             preferred_element_type=jnp.float32)
        oh = (bc == lane_iota).astype(jnp.float32)
        buf[j, pl.ds(s * 128, 128), :] = oh

    @pl.when(i < _TGRID - 1)
    def _():
        pltpu.make_async_copy(buf.at[j],
                              out_hbm.at[pl.ds(i * _BLKR, _BLKR), :],
                              sems.at[j]).start()

    @pl.when(i == _TGRID - 1)
    def _():
        jl = (_TGRID - 1) % _NBUF
        tail = pltpu.make_async_copy(
            buf.at[jl, pl.ds(0, _TAIL), :],
            out_hbm.at[pl.ds((_TGRID - 1) * _BLKR, _TAIL), :],
            sems.at[jl])
        tail.start()
        tail.wait()
        for k in range(_NBUF):
            if k != jl:
                pltpu.make_async_copy(buf.at[k],
                                      out_hbm.at[pl.ds(0, _BLKR), :],
                                      sems.at[k]).wait()


def kernel(zz, zmap):
    zz = zz.reshape(-1).astype(jnp.int32)
    zmap_p = jnp.zeros((128,), jnp.int32).at[:_NZ].set(zmap.astype(jnp.int32))
    idx = _sc_gather(zz, zmap_p)
    out = pl.pallas_call(
        _tc_body,
        grid=(_TGRID,),
        in_specs=[pl.BlockSpec((_BLKR // 128, 128), lambda i: (i, 0))],
        out_specs=pl.BlockSpec(memory_space=pltpu.HBM),
        out_shape=jax.ShapeDtypeStruct((_NROWS, _NZ), jnp.float32),
        scratch_shapes=[
            pltpu.VMEM((_NBUF, _BLKR, _NZ), jnp.float32),
            pltpu.SemaphoreType.DMA((_NBUF,)),
        ],
    )(idx.reshape(_NPAD // 128, 128))
    return out
